# Initial kernel scaffold; baseline (speedup 1.0000x reference)
import jax
import jax.numpy as jnp
from jax.experimental import pallas as pl


def kernel(x1, edge_index1, x2, edge_index2, x3, edge_index3, W11, b11, W12, b12, W13, b13, W21, b21, W22, b22, W23, b23, W31, b31, W32, b32, W33, b33, fcW, fcb):
    raise NotImplementedError("write your pallas kernel here")


# speedup vs baseline: 10.4267x; 10.4267x over previous
import functools

import jax
import jax.numpy as jnp
from jax import lax
from jax.experimental import pallas as pl
from jax.experimental.pallas import tpu as pltpu
from jax.experimental.pallas import tpu_sc as plsc

NC = 2
NS = 16
LANES = 16
CHUNK = 128
BN = 4096
PR = BN // 8


def _mesh():
    return plsc.VectorSubcoreMesh(core_axis_name="c", subcore_axis_name="s")


_SC_PARAMS = pltpu.CompilerParams(
    needs_layout_passes=False, use_tc_tiling_on_sc=False)


def _sc_degree(d_idx, N, deg_rows):
    E = d_idx.shape[0]
    NW = NC * NS
    assert E % CHUNK == 0
    T = E // CHUNK
    cpw, extra = divmod(T, NW)
    BATCH = 30
    nbatch, brem = divmod(cpw, BATCH)
    region = deg_rows * 128
    assert N <= region

    @functools.partial(
        pl.kernel,
        out_type=jax.ShapeDtypeStruct((NW, deg_rows, 128), jnp.float32),
        mesh=_mesh(),
        compiler_params=_SC_PARAMS,
        scratch_types=[
            pltpu.VMEM((deg_rows, 128), jnp.float32),
            pltpu.VMEM((BATCH * CHUNK,), jnp.int32),
        ],
    )
    def deg_kernel(d_hbm, out_hbm, deg_v, idx_v):
        c = lax.axis_index("c")
        s = lax.axis_index("s")
        wid = s * NC + c
        zeros16 = jnp.zeros((LANES,), jnp.float32)
        ones16 = jnp.ones((LANES,), jnp.float32)

        def zero_body(i, _):
            deg_v[i // 8, pl.ds((i % 8) * LANES, LANES)] = zeros16
            return 0

        lax.fori_loop(0, region // LANES, zero_body, 0, unroll=8)

        def count(nedge):
            def vec_body(j, _):
                idx = idx_v[pl.ds(j * LANES, LANES)]
                plsc.addupdate_scatter(
                    deg_v,
                    [lax.shift_right_logical(idx, 7),
                     jnp.bitwise_and(idx, 127)],
                    ones16)
                return 0

            lax.fori_loop(0, nedge // LANES, vec_body, 0, unroll=4)

        base = wid * cpw * CHUNK

        def batch_body(i, _):
            pltpu.sync_copy(
                d_hbm.at[pl.ds(base + i * BATCH * CHUNK, BATCH * CHUNK)],
                idx_v)
            count(BATCH * CHUNK)
            return 0

        lax.fori_loop(0, nbatch, batch_body, 0)
        if brem:
            pltpu.sync_copy(
                d_hbm.at[pl.ds(base + nbatch * BATCH * CHUNK, brem * CHUNK)],
                idx_v.at[pl.ds(0, brem * CHUNK)])
            count(brem * CHUNK)
        if extra:
            @pl.when(wid < extra)
            def _():
                pltpu.sync_copy(
                    d_hbm.at[pl.ds((NW * cpw + wid) * CHUNK, CHUNK)],
                    idx_v.at[pl.ds(0, CHUNK)])
                count(CHUNK)
        pltpu.sync_copy(deg_v, out_hbm.at[wid])

    return deg_kernel(d_idx)


def _sc_edge(y_flat, s_idx, d_idx, N2, H):
    E = s_idx.shape[0]
    assert E % CHUNK == 0
    T = E // CHUNK
    cpt, extra = divmod(T, NS)
    rpt = ((N2 // NS) + 7) // 8 * 8
    last_rows = N2 - rpt * (NS - 1)
    assert last_rows > 0 and last_rows % 8 == 0

    @functools.partial(
        pl.kernel,
        out_type=jax.ShapeDtypeStruct((NC * N2, H), jnp.float32),
        mesh=_mesh(),
        compiler_params=_SC_PARAMS,
        scratch_types=[
            pltpu.VMEM_SHARED((N2, H), jnp.float32),
            pltpu.VMEM((CHUNK,), jnp.int32),
            pltpu.VMEM((CHUNK,), jnp.int32),
            pltpu.VMEM((CHUNK, H), jnp.float32),
            pltpu.SemaphoreType.DMA,
        ],
    )
    def edge_kernel(y_hbm, s_hbm, d_hbm, z_hbm, z_sh, sbuf, dbuf, rows, sem):
        c = lax.axis_index("c")
        t = lax.axis_index("s")
        cn = c * N2
        r0 = t * rpt
        @pl.when(t < NS - 1)
        def _():
            pltpu.sync_copy(y_hbm.at[pl.ds(cn + r0, rpt)],
                            z_sh.at[pl.ds(r0, rpt)])

        @pl.when(t == NS - 1)
        def _():
            pltpu.sync_copy(y_hbm.at[pl.ds(cn + (NS - 1) * rpt, last_rows)],
                            z_sh.at[pl.ds((NS - 1) * rpt, last_rows)])

        plsc.subcore_barrier()

        cn16 = jnp.zeros((LANES,), jnp.int32) + cn

        def step(b):
            pltpu.sync_copy(s_hbm.at[b], sbuf)
            pltpu.sync_copy(d_hbm.at[b], dbuf)
            def bias(j, _):
                sbuf[pl.ds(j * LANES, LANES)] = (
                    sbuf[pl.ds(j * LANES, LANES)] + cn16)
                return 0

            lax.fori_loop(0, CHUNK // LANES, bias, 0, unroll=8)
            pltpu.async_copy(y_hbm.at[sbuf], rows, sem).wait()
            pltpu.sync_copy(rows, z_sh.at[dbuf], add=True)

        ebase = t * cpt * CHUNK

        def body(i, _):
            step(pl.ds(ebase + i * CHUNK, CHUNK))
            return 0

        lax.fori_loop(0, cpt, body, 0)
        if extra:
            @pl.when(t < extra)
            def _():
                step(pl.ds((NS * cpt + t) * CHUNK, CHUNK))

        plsc.subcore_barrier()

        @pl.when(t < NS - 1)
        def _():
            pltpu.sync_copy(z_sh.at[pl.ds(r0, rpt)],
                            z_hbm.at[pl.ds(cn + r0, rpt)])

        @pl.when(t == NS - 1)
        def _():
            pltpu.sync_copy(z_sh.at[pl.ds((NS - 1) * rpt, last_rows)],
                            z_hbm.at[pl.ds(cn + (NS - 1) * rpt, last_rows)])

    return edge_kernel(y_flat, s_idx, d_idx)


def _pack(y, H):
    y3 = y.reshape(PR, 8, 2 * H)
    h0 = y3[:, :, :H].reshape(PR, 8 * H)
    h1 = y3[:, :, H:].reshape(PR, 8 * H)
    return h0, h1


def _unpack(z_ref, H):
    z0 = z_ref[0].reshape(PR, 8, H)
    z1 = z_ref[1].reshape(PR, 8, H)
    return jnp.concatenate([z0, z1], axis=2).reshape(PR * 8, 2 * H)


def _tc_stage1(degp, x, W, N, NB8, H):
    NW = degp.shape[0]
    degp = degp.reshape(NW, degp.shape[1] * 128)
    Fin = x.shape[1]
    grid = pl.cdiv(N, BN)
    assert degp.shape[1] == grid * BN

    def body(degp_ref, x_ref, w_ref, y_ref, dinv_ref):
        dinv = lax.rsqrt(1.0 + jnp.sum(degp_ref[...], axis=0))
        xw = jnp.dot(x_ref[...], w_ref[...],
                     preferred_element_type=jnp.float32)
        y = xw * dinv[:, None]
        h0, h1 = _pack(y, H)
        y_ref[0] = h0
        y_ref[1] = h1
        dinv_ref[...] = dinv

    return pl.pallas_call(
        body,
        grid=(grid,),
        in_specs=[
            pl.BlockSpec((NW, BN), lambda i: (0, i)),
            pl.BlockSpec((BN, Fin), lambda i: (i, 0)),
            pl.BlockSpec((Fin, 2 * H), lambda i: (0, 0)),
        ],
        out_specs=[
            pl.BlockSpec((NC, PR, 128), lambda i: (0, i, 0)),
            pl.BlockSpec((BN,), lambda i: (i,)),
        ],
        out_shape=[
            jax.ShapeDtypeStruct((NC, NB8, 128), jnp.float32),
            jax.ShapeDtypeStruct((N,), jnp.float32),
        ],
    )(degp, x, W)


def _tc_mid(z, dinv, b, W, N, H):
    grid = pl.cdiv(N, BN)
    NB8 = z.shape[1]

    def body(z_ref, dinv_ref, b_ref, w_ref, y_ref):
        dinv = dinv_ref[...]
        zf = _unpack(z_ref, H)
        h = jnp.maximum(zf * dinv[:, None] + b_ref[...][None, :], 0.0)
        y = jnp.dot(h, w_ref[...],
                    preferred_element_type=jnp.float32) * dinv[:, None]
        h0, h1 = _pack(y, H)
        y_ref[0] = h0
        y_ref[1] = h1

    return pl.pallas_call(
        body,
        grid=(grid,),
        in_specs=[
            pl.BlockSpec((NC, PR, 128), lambda i: (0, i, 0)),
            pl.BlockSpec((BN,), lambda i: (i,)),
            pl.BlockSpec((2 * H,), lambda i: (0,)),
            pl.BlockSpec((2 * H, 2 * H), lambda i: (0, 0)),
        ],
        out_specs=pl.BlockSpec((NC, PR, 128), lambda i: (0, i, 0)),
        out_shape=jax.ShapeDtypeStruct((NC, NB8, 128), jnp.float32),
    )(z, dinv, b, W)


def _tc_final(z, dinv, b, fcW, fcb, N, H):
    grid = pl.cdiv(N, BN)

    def body(z_ref, dinv_ref, b_ref, w_ref, fcb_ref, o_ref):
        dinv = dinv_ref[...]
        zf = _unpack(z_ref, H)
        h = jnp.maximum(zf * dinv[:, None] + b_ref[...][None, :], 0.0)
        o = jnp.dot(h, w_ref[...], preferred_element_type=jnp.float32)
        o_ref[...] = jnp.reshape(o, (o.shape[0],)) + fcb_ref[0]

    return pl.pallas_call(
        body,
        grid=(grid,),
        in_specs=[
            pl.BlockSpec((NC, PR, 128), lambda i: (0, i, 0)),
            pl.BlockSpec((BN,), lambda i: (i,)),
            pl.BlockSpec((2 * H,), lambda i: (0,)),
            pl.BlockSpec((2 * H, 1), lambda i: (0, 0)),
            pl.BlockSpec((1,), lambda i: (0,)),
        ],
        out_specs=pl.BlockSpec((BN,), lambda i: (i,)),
        out_shape=jax.ShapeDtypeStruct((N,), jnp.float32),
    )(z, dinv, b, fcW, fcb)


def kernel(x1, edge_index1, x2, edge_index2, x3, edge_index3,
           W11, b11, W12, b12, W13, b13,
           W21, b21, W22, b22, W23, b23,
           W31, b31, W32, b32, W33, b33,
           fcW, fcb):
    N = x1.shape[0]
    H = W11.shape[1] // NC
    grid = pl.cdiv(N, BN)
    deg_rows = grid * (BN // 128)
    NB8 = ((N + 7) // 8 + 7) // 8 * 8
    N2 = NB8 * (128 // H)

    s_idx = edge_index1[0]
    d_idx = edge_index1[1]

    def edge(yp):
        zf = _sc_edge(yp.reshape(NC * N2, H), s_idx, d_idx, N2, H)
        return zf.reshape(NC, NB8, 128)

    degp = _sc_degree(d_idx, N, deg_rows)
    y1, dinv = _tc_stage1(degp, x1, W11, N, NB8, H)
    z1 = edge(y1)
    y2 = _tc_mid(z1, dinv, b11, W12, N, H)
    z2 = edge(y2)
    y3 = _tc_mid(z2, dinv, b12, W13, N, H)
    z3 = edge(y3)
    out = _tc_final(z3, dinv, b13, fcW, fcb, N, H)
    return out.reshape(N, 1)


# speedup vs baseline: 33.4738x; 3.2104x over previous
import functools

import jax
import jax.numpy as jnp
from jax import lax
from jax.experimental import pallas as pl
from jax.experimental.pallas import tpu as pltpu
from jax.experimental.pallas import tpu_sc as plsc

NC = 2
NS = 16
LANES = 16
CHUNK = 128
BN = 4096
PR = BN // 8


def _mesh():
    return plsc.VectorSubcoreMesh(core_axis_name="c", subcore_axis_name="s")


_SC_PARAMS = pltpu.CompilerParams(
    needs_layout_passes=False, use_tc_tiling_on_sc=False)


def _sc_degree(d_idx, N, deg_rows):
    E = d_idx.shape[0]
    NW = NC * NS
    assert E % CHUNK == 0
    T = E // CHUNK
    cpw, extra = divmod(T, NW)
    BATCH = 30
    nbatch, brem = divmod(cpw, BATCH)
    region = deg_rows * 128
    assert N <= region

    @functools.partial(
        pl.kernel,
        out_type=jax.ShapeDtypeStruct((NW, deg_rows, 128), jnp.float32),
        mesh=_mesh(),
        compiler_params=_SC_PARAMS,
        scratch_types=[
            pltpu.VMEM((deg_rows, 128), jnp.float32),
            pltpu.VMEM((BATCH * CHUNK,), jnp.int32),
        ],
    )
    def deg_kernel(d_hbm, out_hbm, deg_v, idx_v):
        c = lax.axis_index("c")
        s = lax.axis_index("s")
        wid = s * NC + c
        zeros16 = jnp.zeros((LANES,), jnp.float32)
        ones16 = jnp.ones((LANES,), jnp.float32)

        def zero_body(i, _):
            deg_v[i // 8, pl.ds((i % 8) * LANES, LANES)] = zeros16
            return 0

        lax.fori_loop(0, region // LANES, zero_body, 0, unroll=8)

        def count(nedge):
            def vec_body(j, _):
                idx = idx_v[pl.ds(j * LANES, LANES)]
                plsc.addupdate_scatter(
                    deg_v,
                    [lax.shift_right_logical(idx, 7),
                     jnp.bitwise_and(idx, 127)],
                    ones16)
                return 0

            lax.fori_loop(0, nedge // LANES, vec_body, 0, unroll=4)

        base = wid * cpw * CHUNK

        def batch_body(i, _):
            pltpu.sync_copy(
                d_hbm.at[pl.ds(base + i * BATCH * CHUNK, BATCH * CHUNK)],
                idx_v)
            count(BATCH * CHUNK)
            return 0

        lax.fori_loop(0, nbatch, batch_body, 0)
        if brem:
            pltpu.sync_copy(
                d_hbm.at[pl.ds(base + nbatch * BATCH * CHUNK, brem * CHUNK)],
                idx_v.at[pl.ds(0, brem * CHUNK)])
            count(brem * CHUNK)
        if extra:
            @pl.when(wid < extra)
            def _():
                pltpu.sync_copy(
                    d_hbm.at[pl.ds((NW * cpw + wid) * CHUNK, CHUNK)],
                    idx_v.at[pl.ds(0, CHUNK)])
                count(CHUNK)
        pltpu.sync_copy(deg_v, out_hbm.at[wid])

    return deg_kernel(d_idx)


def _sc_edge(y_flat, s_idx, d_idx, N2, H):
    E = s_idx.shape[0]
    assert E % CHUNK == 0
    T = E // CHUNK
    cpt, extra = divmod(T, NS)
    rpt = ((N2 // NS) + 7) // 8 * 8
    last_rows = N2 - rpt * (NS - 1)
    assert last_rows > 0 and last_rows % 8 == 0
    D = 6
    pairs = cpt // (2 * D)
    tail = cpt - pairs * 2 * D

    idx_t = pltpu.VMEM((CHUNK,), jnp.int32)
    row_t = pltpu.VMEM((CHUNK, H), jnp.float32)
    scratch = ([pltpu.VMEM_SHARED((N2, H), jnp.float32)]
               + [idx_t] * (4 * D) + [row_t] * (2 * D)
               + [pltpu.SemaphoreType.DMA] * 3)

    @functools.partial(
        pl.kernel,
        out_type=jax.ShapeDtypeStruct((NC * N2, H), jnp.float32),
        mesh=_mesh(),
        compiler_params=_SC_PARAMS,
        scratch_types=scratch,
    )
    def edge_kernel(y_hbm, s_hbm, d_hbm, z_hbm, z_sh, *sc):
        sA = sc[0:D]
        dA = sc[D:2 * D]
        sB = sc[2 * D:3 * D]
        dB = sc[3 * D:4 * D]
        rA = sc[4 * D:5 * D]
        rB = sc[5 * D:6 * D]
        semiA, semiB, semg = sc[6 * D:6 * D + 3]

        c = lax.axis_index("c")
        t = lax.axis_index("s")
        cn = c * N2
        yc = y_hbm.at[pl.ds(cn, N2)]
        r0 = t * rpt
        @pl.when(t < NS - 1)
        def _():
            pltpu.sync_copy(y_hbm.at[pl.ds(cn + r0, rpt)],
                            z_sh.at[pl.ds(r0, rpt)])

        @pl.when(t == NS - 1)
        def _():
            pltpu.sync_copy(y_hbm.at[pl.ds(cn + (NS - 1) * rpt, last_rows)],
                            z_sh.at[pl.ds((NS - 1) * rpt, last_rows)])

        plsc.subcore_barrier()

        ebase = t * cpt * CHUNK

        def fire_idx(base, sb, db, sem):
            for k in range(D):
                pltpu.async_copy(
                    s_hbm.at[pl.ds(base + k * CHUNK, CHUNK)], sb[k], sem)
                pltpu.async_copy(
                    d_hbm.at[pl.ds(base + k * CHUNK, CHUNK)], db[k], sem)

        def wait_idx(sb, db, sem):
            for k in range(D):
                pltpu.make_async_copy(
                    s_hbm.at[pl.ds(0, CHUNK)], sb[k], sem).wait()
                pltpu.make_async_copy(
                    s_hbm.at[pl.ds(0, CHUNK)], db[k], sem).wait()

        def do_group(sb, db, rr):
            for k in range(D):
                pltpu.async_copy(yc.at[sb[k]], rr[k], semg)
            for k in range(D):
                pltpu.make_async_copy(
                    yc.at[pl.ds(0, CHUNK)], rr[k], semg).wait()
            for k in range(D):
                pltpu.sync_copy(rr[k], z_sh.at[db[k]], add=True)

        fire_idx(ebase, sA, dA, semiA)

        def pair_body(p, _):
            gA = 2 * p
            wait_idx(sA, dA, semiA)
            fire_idx(ebase + (gA + 1) * D * CHUNK, sB, dB, semiB)
            do_group(sA, dA, rA)
            wait_idx(sB, dB, semiB)

            @pl.when(p + 1 < pairs)
            def _():
                fire_idx(ebase + (gA + 2) * D * CHUNK, sA, dA, semiA)

            do_group(sB, dB, rB)
            return 0

        lax.fori_loop(0, pairs, pair_body, 0)

        def step(b):
            pltpu.sync_copy(s_hbm.at[b], sA[0])
            pltpu.sync_copy(d_hbm.at[b], dA[0])
            pltpu.async_copy(yc.at[sA[0]], rA[0], semg).wait()
            pltpu.sync_copy(rA[0], z_sh.at[dA[0]], add=True)

        for i in range(tail):
            step(pl.ds(ebase + (pairs * 2 * D + i) * CHUNK, CHUNK))
        if extra:
            @pl.when(t < extra)
            def _():
                step(pl.ds((NS * cpt + t) * CHUNK, CHUNK))

        plsc.subcore_barrier()

        @pl.when(t < NS - 1)
        def _():
            pltpu.sync_copy(z_sh.at[pl.ds(r0, rpt)],
                            z_hbm.at[pl.ds(cn + r0, rpt)])

        @pl.when(t == NS - 1)
        def _():
            pltpu.sync_copy(z_sh.at[pl.ds((NS - 1) * rpt, last_rows)],
                            z_hbm.at[pl.ds(cn + (NS - 1) * rpt, last_rows)])

    return edge_kernel(y_flat, s_idx, d_idx)


def _pack(y, H):
    y3 = y.reshape(PR, 8, 2 * H)
    h0 = y3[:, :, :H].reshape(PR, 8 * H)
    h1 = y3[:, :, H:].reshape(PR, 8 * H)
    return h0, h1


def _unpack(z_ref, H):
    z0 = z_ref[0].reshape(PR, 8, H)
    z1 = z_ref[1].reshape(PR, 8, H)
    return jnp.concatenate([z0, z1], axis=2).reshape(PR * 8, 2 * H)


def _tc_stage1(degp, x, W, N, NB8, H):
    NW = degp.shape[0]
    degp = degp.reshape(NW, degp.shape[1] * 128)
    Fin = x.shape[1]
    grid = pl.cdiv(N, BN)
    assert degp.shape[1] == grid * BN

    def body(degp_ref, x_ref, w_ref, y_ref, dinv_ref):
        dinv = lax.rsqrt(1.0 + jnp.sum(degp_ref[...], axis=0))
        xw = jnp.dot(x_ref[...], w_ref[...],
                     preferred_element_type=jnp.float32)
        y = xw * dinv[:, None]
        h0, h1 = _pack(y, H)
        y_ref[0] = h0
        y_ref[1] = h1
        dinv_ref[...] = dinv

    return pl.pallas_call(
        body,
        grid=(grid,),
        in_specs=[
            pl.BlockSpec((NW, BN), lambda i: (0, i)),
            pl.BlockSpec((BN, Fin), lambda i: (i, 0)),
            pl.BlockSpec((Fin, 2 * H), lambda i: (0, 0)),
        ],
        out_specs=[
            pl.BlockSpec((NC, PR, 128), lambda i: (0, i, 0)),
            pl.BlockSpec((BN,), lambda i: (i,)),
        ],
        out_shape=[
            jax.ShapeDtypeStruct((NC, NB8, 128), jnp.float32),
            jax.ShapeDtypeStruct((N,), jnp.float32),
        ],
    )(degp, x, W)


def _tc_mid(z, dinv, b, W, N, H):
    grid = pl.cdiv(N, BN)
    NB8 = z.shape[1]

    def body(z_ref, dinv_ref, b_ref, w_ref, y_ref):
        dinv = dinv_ref[...]
        zf = _unpack(z_ref, H)
        h = jnp.maximum(zf * dinv[:, None] + b_ref[...][None, :], 0.0)
        y = jnp.dot(h, w_ref[...],
                    preferred_element_type=jnp.float32) * dinv[:, None]
        h0, h1 = _pack(y, H)
        y_ref[0] = h0
        y_ref[1] = h1

    return pl.pallas_call(
        body,
        grid=(grid,),
        in_specs=[
            pl.BlockSpec((NC, PR, 128), lambda i: (0, i, 0)),
            pl.BlockSpec((BN,), lambda i: (i,)),
            pl.BlockSpec((2 * H,), lambda i: (0,)),
            pl.BlockSpec((2 * H, 2 * H), lambda i: (0, 0)),
        ],
        out_specs=pl.BlockSpec((NC, PR, 128), lambda i: (0, i, 0)),
        out_shape=jax.ShapeDtypeStruct((NC, NB8, 128), jnp.float32),
    )(z, dinv, b, W)


def _tc_final(z, dinv, b, fcW, fcb, N, H):
    grid = pl.cdiv(N, BN)

    def body(z_ref, dinv_ref, b_ref, w_ref, fcb_ref, o_ref):
        dinv = dinv_ref[...]
        zf = _unpack(z_ref, H)
        h = jnp.maximum(zf * dinv[:, None] + b_ref[...][None, :], 0.0)
        o = jnp.dot(h, w_ref[...], preferred_element_type=jnp.float32)
        o_ref[...] = jnp.reshape(o, (o.shape[0],)) + fcb_ref[0]

    return pl.pallas_call(
        body,
        grid=(grid,),
        in_specs=[
            pl.BlockSpec((NC, PR, 128), lambda i: (0, i, 0)),
            pl.BlockSpec((BN,), lambda i: (i,)),
            pl.BlockSpec((2 * H,), lambda i: (0,)),
            pl.BlockSpec((2 * H, 1), lambda i: (0, 0)),
            pl.BlockSpec((1,), lambda i: (0,)),
        ],
        out_specs=pl.BlockSpec((BN,), lambda i: (i,)),
        out_shape=jax.ShapeDtypeStruct((N,), jnp.float32),
    )(z, dinv, b, fcW, fcb)


def kernel(x1, edge_index1, x2, edge_index2, x3, edge_index3,
           W11, b11, W12, b12, W13, b13,
           W21, b21, W22, b22, W23, b23,
           W31, b31, W32, b32, W33, b33,
           fcW, fcb):
    N = x1.shape[0]
    H = W11.shape[1] // NC
    grid = pl.cdiv(N, BN)
    deg_rows = grid * (BN // 128)
    NB8 = ((N + 7) // 8 + 7) // 8 * 8
    N2 = NB8 * (128 // H)

    s_idx = edge_index1[0]
    d_idx = edge_index1[1]

    def edge(yp):
        zf = _sc_edge(yp.reshape(NC * N2, H), s_idx, d_idx, N2, H)
        return zf.reshape(NC, NB8, 128)

    degp = _sc_degree(d_idx, N, deg_rows)
    y1, dinv = _tc_stage1(degp, x1, W11, N, NB8, H)
    z1 = edge(y1)
    y2 = _tc_mid(z1, dinv, b11, W12, N, H)
    z2 = edge(y2)
    y3 = _tc_mid(z2, dinv, b12, W13, N, H)
    z3 = edge(y3)
    out = _tc_final(z3, dinv, b13, fcW, fcb, N, H)
    return out.reshape(N, 1)


# speedup vs baseline: 43.9767x; 1.3138x over previous
import functools

import jax
import jax.numpy as jnp
from jax import lax
from jax.experimental import pallas as pl
from jax.experimental.pallas import tpu as pltpu
from jax.experimental.pallas import tpu_sc as plsc

NC = 2
NS = 16
LANES = 16
CHUNK = 128
BN = 4096
PR = BN // 8


def _mesh():
    return plsc.VectorSubcoreMesh(core_axis_name="c", subcore_axis_name="s")


_SC_PARAMS = pltpu.CompilerParams(
    needs_layout_passes=False, use_tc_tiling_on_sc=False)


def _sc_degree(d_idx, N, deg_rows):
    E = d_idx.shape[0]
    NW = NC * NS
    assert E % CHUNK == 0
    T = E // CHUNK
    cpw, extra = divmod(T, NW)
    BATCH = 30
    nbatch, brem = divmod(cpw, BATCH)
    region = deg_rows * 128
    assert N <= region

    @functools.partial(
        pl.kernel,
        out_type=jax.ShapeDtypeStruct((NW, deg_rows, 128), jnp.float32),
        mesh=_mesh(),
        compiler_params=_SC_PARAMS,
        scratch_types=[
            pltpu.VMEM((deg_rows, 128), jnp.float32),
            pltpu.VMEM((BATCH * CHUNK,), jnp.int32),
        ],
    )
    def deg_kernel(d_hbm, out_hbm, deg_v, idx_v):
        c = lax.axis_index("c")
        s = lax.axis_index("s")
        wid = s * NC + c
        zeros16 = jnp.zeros((LANES,), jnp.float32)
        ones16 = jnp.ones((LANES,), jnp.float32)

        def zero_body(i, _):
            deg_v[i // 8, pl.ds((i % 8) * LANES, LANES)] = zeros16
            return 0

        lax.fori_loop(0, region // LANES, zero_body, 0, unroll=8)

        def count(nedge):
            def vec_body(j, _):
                idx = idx_v[pl.ds(j * LANES, LANES)]
                plsc.addupdate_scatter(
                    deg_v,
                    [lax.shift_right_logical(idx, 7),
                     jnp.bitwise_and(idx, 127)],
                    ones16)
                return 0

            lax.fori_loop(0, nedge // LANES, vec_body, 0, unroll=4)

        base = wid * cpw * CHUNK

        def batch_body(i, _):
            pltpu.sync_copy(
                d_hbm.at[pl.ds(base + i * BATCH * CHUNK, BATCH * CHUNK)],
                idx_v)
            count(BATCH * CHUNK)
            return 0

        lax.fori_loop(0, nbatch, batch_body, 0)
        if brem:
            pltpu.sync_copy(
                d_hbm.at[pl.ds(base + nbatch * BATCH * CHUNK, brem * CHUNK)],
                idx_v.at[pl.ds(0, brem * CHUNK)])
            count(brem * CHUNK)
        if extra:
            @pl.when(wid < extra)
            def _():
                pltpu.sync_copy(
                    d_hbm.at[pl.ds((NW * cpw + wid) * CHUNK, CHUNK)],
                    idx_v.at[pl.ds(0, CHUNK)])
                count(CHUNK)
        pltpu.sync_copy(deg_v, out_hbm.at[wid])

    return deg_kernel(d_idx)


def _sc_edge(y_flat, s_idx, d_idx, N2, H):
    E = s_idx.shape[0]
    assert E % CHUNK == 0
    T = E // CHUNK
    cpt, extra = divmod(T, NS)
    rpt = ((N2 // NS) + 7) // 8 * 8
    last_rows = N2 - rpt * (NS - 1)
    assert last_rows > 0 and last_rows % 8 == 0
    D = 6
    pairs = cpt // (2 * D)
    tail = cpt - pairs * 2 * D

    idx_t = pltpu.VMEM((CHUNK,), jnp.int32)
    row_t = pltpu.VMEM((CHUNK, H), jnp.float32)
    scratch = ([pltpu.VMEM_SHARED((N2, H), jnp.float32)]
               + [idx_t] * (4 * D) + [row_t] * (2 * D)
               + [pltpu.SemaphoreType.DMA] * (4 + 2 * D))

    @functools.partial(
        pl.kernel,
        out_type=jax.ShapeDtypeStruct((NC * N2, H), jnp.float32),
        mesh=_mesh(),
        compiler_params=_SC_PARAMS,
        scratch_types=scratch,
    )
    def edge_kernel(y_hbm, s_hbm, d_hbm, z_hbm, z_sh, *sc):
        sA = sc[0:D]
        dA = sc[D:2 * D]
        sB = sc[2 * D:3 * D]
        dB = sc[3 * D:4 * D]
        rA = sc[4 * D:5 * D]
        rB = sc[5 * D:6 * D]
        semiA, semiB, scsA, scsB = sc[6 * D:6 * D + 4]
        gsA = sc[6 * D + 4:7 * D + 4]
        gsB = sc[7 * D + 4:8 * D + 4]

        c = lax.axis_index("c")
        t = lax.axis_index("s")
        cn = c * N2
        yc = y_hbm.at[pl.ds(cn, N2)]
        r0 = t * rpt
        @pl.when(t < NS - 1)
        def _():
            pltpu.sync_copy(y_hbm.at[pl.ds(cn + r0, rpt)],
                            z_sh.at[pl.ds(r0, rpt)])

        @pl.when(t == NS - 1)
        def _():
            pltpu.sync_copy(y_hbm.at[pl.ds(cn + (NS - 1) * rpt, last_rows)],
                            z_sh.at[pl.ds((NS - 1) * rpt, last_rows)])

        plsc.subcore_barrier()

        ebase = t * cpt * CHUNK

        def fire_idx(base, sb, db, sem):
            for k in range(D):
                pltpu.async_copy(
                    s_hbm.at[pl.ds(base + k * CHUNK, CHUNK)], sb[k], sem)
                pltpu.async_copy(
                    d_hbm.at[pl.ds(base + k * CHUNK, CHUNK)], db[k], sem)

        def wait_idx(sb, db, sem):
            for k in range(D):
                pltpu.make_async_copy(
                    s_hbm.at[pl.ds(0, CHUNK)], sb[k], sem).wait()
                pltpu.make_async_copy(
                    s_hbm.at[pl.ds(0, CHUNK)], db[k], sem).wait()

        def do_group(sb, db, rr, gs, scs):
            for k in range(D):
                pltpu.async_copy(yc.at[sb[k]], rr[k], gs[k])
            for k in range(D):
                pltpu.make_async_copy(
                    yc.at[pl.ds(0, CHUNK)], rr[k], gs[k]).wait()
                pltpu.async_copy(rr[k], z_sh.at[db[k]], scs, add=True)
            for k in range(D):
                pltpu.make_async_copy(rr[k], z_sh.at[db[k]], scs).wait()

        fire_idx(ebase, sA, dA, semiA)

        def pair_body(p, _):
            gA = 2 * p
            wait_idx(sA, dA, semiA)
            fire_idx(ebase + (gA + 1) * D * CHUNK, sB, dB, semiB)
            do_group(sA, dA, rA, gsA, scsA)
            wait_idx(sB, dB, semiB)

            @pl.when(p + 1 < pairs)
            def _():
                fire_idx(ebase + (gA + 2) * D * CHUNK, sA, dA, semiA)

            do_group(sB, dB, rB, gsB, scsB)
            return 0

        lax.fori_loop(0, pairs, pair_body, 0)

        def step(b):
            pltpu.sync_copy(s_hbm.at[b], sA[0])
            pltpu.sync_copy(d_hbm.at[b], dA[0])
            pltpu.async_copy(yc.at[sA[0]], rA[0], gsA[0]).wait()
            pltpu.sync_copy(rA[0], z_sh.at[dA[0]], add=True)

        for i in range(tail):
            step(pl.ds(ebase + (pairs * 2 * D + i) * CHUNK, CHUNK))
        if extra:
            @pl.when(t < extra)
            def _():
                step(pl.ds((NS * cpt + t) * CHUNK, CHUNK))

        plsc.subcore_barrier()

        @pl.when(t < NS - 1)
        def _():
            pltpu.sync_copy(z_sh.at[pl.ds(r0, rpt)],
                            z_hbm.at[pl.ds(cn + r0, rpt)])

        @pl.when(t == NS - 1)
        def _():
            pltpu.sync_copy(z_sh.at[pl.ds((NS - 1) * rpt, last_rows)],
                            z_hbm.at[pl.ds(cn + (NS - 1) * rpt, last_rows)])

    return edge_kernel(y_flat, s_idx, d_idx)


def _pack(y, H):
    y3 = y.reshape(PR, 8, 2 * H)
    h0 = y3[:, :, :H].reshape(PR, 8 * H)
    h1 = y3[:, :, H:].reshape(PR, 8 * H)
    return h0, h1


def _unpack(z_ref, H):
    z0 = z_ref[0].reshape(PR, 8, H)
    z1 = z_ref[1].reshape(PR, 8, H)
    return jnp.concatenate([z0, z1], axis=2).reshape(PR * 8, 2 * H)


def _tc_stage1(degp, x, W, N, NB8, H):
    NW = degp.shape[0]
    degp = degp.reshape(NW, degp.shape[1] * 128)
    Fin = x.shape[1]
    grid = pl.cdiv(N, BN)
    assert degp.shape[1] == grid * BN

    def body(degp_ref, x_ref, w_ref, y_ref, dinv_ref):
        dinv = lax.rsqrt(1.0 + jnp.sum(degp_ref[...], axis=0))
        xw = jnp.dot(x_ref[...], w_ref[...],
                     preferred_element_type=jnp.float32)
        y = xw * dinv[:, None]
        h0, h1 = _pack(y, H)
        y_ref[0] = h0
        y_ref[1] = h1
        dinv_ref[...] = dinv

    return pl.pallas_call(
        body,
        grid=(grid,),
        in_specs=[
            pl.BlockSpec((NW, BN), lambda i: (0, i)),
            pl.BlockSpec((BN, Fin), lambda i: (i, 0)),
            pl.BlockSpec((Fin, 2 * H), lambda i: (0, 0)),
        ],
        out_specs=[
            pl.BlockSpec((NC, PR, 128), lambda i: (0, i, 0)),
            pl.BlockSpec((BN,), lambda i: (i,)),
        ],
        out_shape=[
            jax.ShapeDtypeStruct((NC, NB8, 128), jnp.float32),
            jax.ShapeDtypeStruct((N,), jnp.float32),
        ],
    )(degp, x, W)


def _tc_mid(z, dinv, b, W, N, H):
    grid = pl.cdiv(N, BN)
    NB8 = z.shape[1]

    def body(z_ref, dinv_ref, b_ref, w_ref, y_ref):
        dinv = dinv_ref[...]
        zf = _unpack(z_ref, H)
        h = jnp.maximum(zf * dinv[:, None] + b_ref[...][None, :], 0.0)
        y = jnp.dot(h, w_ref[...],
                    preferred_element_type=jnp.float32) * dinv[:, None]
        h0, h1 = _pack(y, H)
        y_ref[0] = h0
        y_ref[1] = h1

    return pl.pallas_call(
        body,
        grid=(grid,),
        in_specs=[
            pl.BlockSpec((NC, PR, 128), lambda i: (0, i, 0)),
            pl.BlockSpec((BN,), lambda i: (i,)),
            pl.BlockSpec((2 * H,), lambda i: (0,)),
            pl.BlockSpec((2 * H, 2 * H), lambda i: (0, 0)),
        ],
        out_specs=pl.BlockSpec((NC, PR, 128), lambda i: (0, i, 0)),
        out_shape=jax.ShapeDtypeStruct((NC, NB8, 128), jnp.float32),
    )(z, dinv, b, W)


def _tc_final(z, dinv, b, fcW, fcb, N, H):
    grid = pl.cdiv(N, BN)

    def body(z_ref, dinv_ref, b_ref, w_ref, fcb_ref, o_ref):
        dinv = dinv_ref[...]
        zf = _unpack(z_ref, H)
        h = jnp.maximum(zf * dinv[:, None] + b_ref[...][None, :], 0.0)
        o = jnp.dot(h, w_ref[...], preferred_element_type=jnp.float32)
        o_ref[...] = jnp.reshape(o, (o.shape[0],)) + fcb_ref[0]

    return pl.pallas_call(
        body,
        grid=(grid,),
        in_specs=[
            pl.BlockSpec((NC, PR, 128), lambda i: (0, i, 0)),
            pl.BlockSpec((BN,), lambda i: (i,)),
            pl.BlockSpec((2 * H,), lambda i: (0,)),
            pl.BlockSpec((2 * H, 1), lambda i: (0, 0)),
            pl.BlockSpec((1,), lambda i: (0,)),
        ],
        out_specs=pl.BlockSpec((BN,), lambda i: (i,)),
        out_shape=jax.ShapeDtypeStruct((N,), jnp.float32),
    )(z, dinv, b, fcW, fcb)


def kernel(x1, edge_index1, x2, edge_index2, x3, edge_index3,
           W11, b11, W12, b12, W13, b13,
           W21, b21, W22, b22, W23, b23,
           W31, b31, W32, b32, W33, b33,
           fcW, fcb):
    N = x1.shape[0]
    H = W11.shape[1] // NC
    grid = pl.cdiv(N, BN)
    deg_rows = grid * (BN // 128)
    NB8 = ((N + 7) // 8 + 7) // 8 * 8
    N2 = NB8 * (128 // H)

    s_idx = edge_index1[0]
    d_idx = edge_index1[1]

    def edge(yp):
        zf = _sc_edge(yp.reshape(NC * N2, H), s_idx, d_idx, N2, H)
        return zf.reshape(NC, NB8, 128)

    degp = _sc_degree(d_idx, N, deg_rows)
    y1, dinv = _tc_stage1(degp, x1, W11, N, NB8, H)
    z1 = edge(y1)
    y2 = _tc_mid(z1, dinv, b11, W12, N, H)
    z2 = edge(y2)
    y3 = _tc_mid(z2, dinv, b12, W13, N, H)
    z3 = edge(y3)
    out = _tc_final(z3, dinv, b13, fcW, fcb, N, H)
    return out.reshape(N, 1)


# speedup vs baseline: 49.2154x; 1.1191x over previous
import functools

import jax
import jax.numpy as jnp
from jax import lax
from jax.experimental import pallas as pl
from jax.experimental.pallas import tpu as pltpu
from jax.experimental.pallas import tpu_sc as plsc

NC = 2
NS = 16
LANES = 16
CHUNK = 128
BN = 4096
PR = BN // 8


def _mesh():
    return plsc.VectorSubcoreMesh(core_axis_name="c", subcore_axis_name="s")


_SC_PARAMS = pltpu.CompilerParams(
    needs_layout_passes=False, use_tc_tiling_on_sc=False)


def _sc_degree(d_idx, N, deg_rows):
    E = d_idx.shape[0]
    NW = NC * NS
    assert E % CHUNK == 0
    T = E // CHUNK
    cpw, extra = divmod(T, NW)
    BATCH = 30
    nbatch, brem = divmod(cpw, BATCH)
    region = deg_rows * 128
    assert N <= region

    @functools.partial(
        pl.kernel,
        out_type=jax.ShapeDtypeStruct((NW, deg_rows, 128), jnp.float32),
        mesh=_mesh(),
        compiler_params=_SC_PARAMS,
        scratch_types=[
            pltpu.VMEM((deg_rows, 128), jnp.float32),
            pltpu.VMEM((BATCH * CHUNK,), jnp.int32),
        ],
    )
    def deg_kernel(d_hbm, out_hbm, deg_v, idx_v):
        c = lax.axis_index("c")
        s = lax.axis_index("s")
        wid = s * NC + c
        zeros16 = jnp.zeros((LANES,), jnp.float32)
        ones16 = jnp.ones((LANES,), jnp.float32)

        def zero_body(i, _):
            deg_v[i // 8, pl.ds((i % 8) * LANES, LANES)] = zeros16
            return 0

        lax.fori_loop(0, region // LANES, zero_body, 0, unroll=8)

        def count(nedge):
            def vec_body(j, _):
                idx = idx_v[pl.ds(j * LANES, LANES)]
                plsc.addupdate_scatter(
                    deg_v,
                    [lax.shift_right_logical(idx, 7),
                     jnp.bitwise_and(idx, 127)],
                    ones16)
                return 0

            lax.fori_loop(0, nedge // LANES, vec_body, 0, unroll=4)

        base = wid * cpw * CHUNK

        def batch_body(i, _):
            pltpu.sync_copy(
                d_hbm.at[pl.ds(base + i * BATCH * CHUNK, BATCH * CHUNK)],
                idx_v)
            count(BATCH * CHUNK)
            return 0

        lax.fori_loop(0, nbatch, batch_body, 0)
        if brem:
            pltpu.sync_copy(
                d_hbm.at[pl.ds(base + nbatch * BATCH * CHUNK, brem * CHUNK)],
                idx_v.at[pl.ds(0, brem * CHUNK)])
            count(brem * CHUNK)
        if extra:
            @pl.when(wid < extra)
            def _():
                pltpu.sync_copy(
                    d_hbm.at[pl.ds((NW * cpw + wid) * CHUNK, CHUNK)],
                    idx_v.at[pl.ds(0, CHUNK)])
                count(CHUNK)
        pltpu.sync_copy(deg_v, out_hbm.at[wid])

    return deg_kernel(d_idx)


def _sc_edge(y_flat, s_idx, d_idx, N2, H):
    E = s_idx.shape[0]
    assert E % CHUNK == 0
    T = E // CHUNK
    cpt, extra = divmod(T, NS)
    rpt = ((N2 // NS) + 7) // 8 * 8
    last_rows = N2 - rpt * (NS - 1)
    assert last_rows > 0 and last_rows % 8 == 0
    D = 6
    pairs = cpt // (2 * D)
    tail = cpt - pairs * 2 * D

    idx_t = pltpu.VMEM((CHUNK,), jnp.int32)
    row_t = pltpu.VMEM((CHUNK, H), jnp.float32)
    scratch = ([pltpu.VMEM_SHARED((N2, H), jnp.float32)]
               + [idx_t] * (4 * D) + [row_t] * (2 * D)
               + [pltpu.SemaphoreType.DMA] * (4 + 2 * D))

    @functools.partial(
        pl.kernel,
        out_type=jax.ShapeDtypeStruct((NC * N2, H), jnp.float32),
        mesh=_mesh(),
        compiler_params=_SC_PARAMS,
        scratch_types=scratch,
    )
    def edge_kernel(y_hbm, s_hbm, d_hbm, z_hbm, z_sh, *sc):
        sA = sc[0:D]
        dA = sc[D:2 * D]
        sB = sc[2 * D:3 * D]
        dB = sc[3 * D:4 * D]
        rA = sc[4 * D:5 * D]
        rB = sc[5 * D:6 * D]
        semiA, semiB, scsA, scsB = sc[6 * D:6 * D + 4]
        gsA = sc[6 * D + 4:7 * D + 4]
        gsB = sc[7 * D + 4:8 * D + 4]

        c = lax.axis_index("c")
        t = lax.axis_index("s")
        cn = c * N2
        yc = y_hbm.at[pl.ds(cn, N2)]
        r0 = t * rpt
        @pl.when(t < NS - 1)
        def _():
            pltpu.sync_copy(y_hbm.at[pl.ds(cn + r0, rpt)],
                            z_sh.at[pl.ds(r0, rpt)])

        @pl.when(t == NS - 1)
        def _():
            pltpu.sync_copy(y_hbm.at[pl.ds(cn + (NS - 1) * rpt, last_rows)],
                            z_sh.at[pl.ds((NS - 1) * rpt, last_rows)])

        plsc.subcore_barrier()

        ebase = t * cpt * CHUNK

        def fire_idx(base, sb, db, sem):
            for k in range(D):
                pltpu.async_copy(
                    s_hbm.at[pl.ds(base + k * CHUNK, CHUNK)], sb[k], sem)
                pltpu.async_copy(
                    d_hbm.at[pl.ds(base + k * CHUNK, CHUNK)], db[k], sem)

        def wait_idx(sb, db, sem):
            for k in range(D):
                pltpu.make_async_copy(
                    s_hbm.at[pl.ds(0, CHUNK)], sb[k], sem).wait()
                pltpu.make_async_copy(
                    s_hbm.at[pl.ds(0, CHUNK)], db[k], sem).wait()

        def do_group(sb, db, rr, gs, scs):
            for k in range(D):
                pltpu.async_copy(yc.at[sb[k]], rr[k], gs[k])
            for k in range(D):
                pltpu.make_async_copy(
                    yc.at[pl.ds(0, CHUNK)], rr[k], gs[k]).wait()
                pltpu.async_copy(rr[k], z_sh.at[db[k]], scs, add=True)
            for k in range(D):
                pltpu.make_async_copy(rr[k], z_sh.at[db[k]], scs).wait()

        fire_idx(ebase, sA, dA, semiA)

        def pair_body(p, _):
            gA = 2 * p
            wait_idx(sA, dA, semiA)
            fire_idx(ebase + (gA + 1) * D * CHUNK, sB, dB, semiB)
            do_group(sA, dA, rA, gsA, scsA)
            wait_idx(sB, dB, semiB)

            @pl.when(p + 1 < pairs)
            def _():
                fire_idx(ebase + (gA + 2) * D * CHUNK, sA, dA, semiA)

            do_group(sB, dB, rB, gsB, scsB)
            return 0

        lax.fori_loop(0, pairs, pair_body, 0)

        def step(b):
            pltpu.sync_copy(s_hbm.at[b], sA[0])
            pltpu.sync_copy(d_hbm.at[b], dA[0])
            pltpu.async_copy(yc.at[sA[0]], rA[0], gsA[0]).wait()
            pltpu.sync_copy(rA[0], z_sh.at[dA[0]], add=True)

        for i in range(tail):
            step(pl.ds(ebase + (pairs * 2 * D + i) * CHUNK, CHUNK))
        if extra:
            @pl.when(t < extra)
            def _():
                step(pl.ds((NS * cpt + t) * CHUNK, CHUNK))

        plsc.subcore_barrier()

        @pl.when(t < NS - 1)
        def _():
            pltpu.sync_copy(z_sh.at[pl.ds(r0, rpt)],
                            z_hbm.at[pl.ds(cn + r0, rpt)])

        @pl.when(t == NS - 1)
        def _():
            pltpu.sync_copy(z_sh.at[pl.ds((NS - 1) * rpt, last_rows)],
                            z_hbm.at[pl.ds(cn + (NS - 1) * rpt, last_rows)])

    return edge_kernel(y_flat, s_idx, d_idx)


def _dvp(dinv):
    rep = jnp.broadcast_to(dinv[:, None], (BN, 32))
    return rep.reshape(PR, 8, 32)[:, :, :16].reshape(PR, 128)


def _tc_stage1(degp, xp, BW1, N, NB8, H):
    NW = degp.shape[0]
    degp = degp.reshape(NW, degp.shape[1] * 128)
    K = xp.shape[1]
    grid = pl.cdiv(N, BN)
    assert degp.shape[1] == grid * BN

    def body(degp_ref, xp_ref, w_ref, y_ref, dinv_ref):
        dinv = lax.rsqrt(1.0 + jnp.sum(degp_ref[...], axis=0))
        dvp = _dvp(dinv)
        Y = jnp.dot(xp_ref[...], w_ref[...],
                    preferred_element_type=jnp.float32)
        y_ref[0] = Y[:, :128] * dvp
        y_ref[1] = Y[:, 128:] * dvp
        dinv_ref[...] = dinv

    return pl.pallas_call(
        body,
        grid=(grid,),
        in_specs=[
            pl.BlockSpec((NW, BN), lambda i: (0, i)),
            pl.BlockSpec((PR, K), lambda i: (i, 0)),
            pl.BlockSpec((K, 256), lambda i: (0, 0)),
        ],
        out_specs=[
            pl.BlockSpec((NC, PR, 128), lambda i: (0, i, 0)),
            pl.BlockSpec((BN,), lambda i: (i,)),
        ],
        out_shape=[
            jax.ShapeDtypeStruct((NC, NB8, 128), jnp.float32),
            jax.ShapeDtypeStruct((N,), jnp.float32),
        ],
    )(degp, xp, BW1)


def _tc_mid(z, dinv, bp, BW, N, H):
    grid = pl.cdiv(N, BN)
    NB8 = z.shape[1]

    def body(z_ref, dinv_ref, bp_ref, w_ref, y_ref):
        dvp = _dvp(dinv_ref[...])
        h0 = jnp.maximum(z_ref[0] * dvp + bp_ref[0][None, :], 0.0)
        h1 = jnp.maximum(z_ref[1] * dvp + bp_ref[1][None, :], 0.0)
        Hp = jnp.concatenate([h0, h1], axis=1)
        Y = jnp.dot(Hp, w_ref[...], preferred_element_type=jnp.float32)
        y_ref[0] = Y[:, :128] * dvp
        y_ref[1] = Y[:, 128:] * dvp

    return pl.pallas_call(
        body,
        grid=(grid,),
        in_specs=[
            pl.BlockSpec((NC, PR, 128), lambda i: (0, i, 0)),
            pl.BlockSpec((BN,), lambda i: (i,)),
            pl.BlockSpec((2, 128), lambda i: (0, 0)),
            pl.BlockSpec((256, 256), lambda i: (0, 0)),
        ],
        out_specs=pl.BlockSpec((NC, PR, 128), lambda i: (0, i, 0)),
        out_shape=jax.ShapeDtypeStruct((NC, NB8, 128), jnp.float32),
    )(z, dinv, bp, BW)


def _tc_final(z, dinv, bp, Bfc, fcb, N, H):
    grid = pl.cdiv(N, BN)

    def body(z_ref, dinv_ref, bp_ref, w_ref, fcb_ref, o_ref):
        dvp = _dvp(dinv_ref[...])
        h0 = jnp.maximum(z_ref[0] * dvp + bp_ref[0][None, :], 0.0)
        h1 = jnp.maximum(z_ref[1] * dvp + bp_ref[1][None, :], 0.0)
        Hp = jnp.concatenate([h0, h1], axis=1)
        o8 = jnp.dot(Hp, w_ref[...], preferred_element_type=jnp.float32)
        o_ref[...] = o8 + fcb_ref[0]

    NB8 = z.shape[1]
    return pl.pallas_call(
        body,
        grid=(grid,),
        in_specs=[
            pl.BlockSpec((NC, PR, 128), lambda i: (0, i, 0)),
            pl.BlockSpec((BN,), lambda i: (i,)),
            pl.BlockSpec((2, 128), lambda i: (0, 0)),
            pl.BlockSpec((256, 8), lambda i: (0, 0)),
            pl.BlockSpec((1,), lambda i: (0,)),
        ],
        out_specs=pl.BlockSpec((PR, 8), lambda i: (i, 0)),
        out_shape=jax.ShapeDtypeStruct((NB8, 8), jnp.float32),
    )(z, dinv, bp, Bfc, fcb)


def kernel(x1, edge_index1, x2, edge_index2, x3, edge_index3,
           W11, b11, W12, b12, W13, b13,
           W21, b21, W22, b22, W23, b23,
           W31, b31, W32, b32, W33, b33,
           fcW, fcb):
    N = x1.shape[0]
    H = W11.shape[1] // NC
    grid = pl.cdiv(N, BN)
    deg_rows = grid * (BN // 128)
    NB8 = ((N + 7) // 8 + 7) // 8 * 8
    N2 = NB8 * (128 // H)

    s_idx = edge_index1[0]
    d_idx = edge_index1[1]

    eye8 = jnp.eye(8, dtype=jnp.float32)

    def bigw(W):
        Wr = W.reshape(W.shape[0], 2, H)
        return jnp.einsum('ab,kcf->akcbf', eye8, Wr).reshape(
            8 * W.shape[0], 256)

    def bpack(b):
        return jnp.tile(b.reshape(2, 1, H), (1, 8, 1)).reshape(2, 128)

    xp = jnp.pad(x1, ((0, N2 - N), (0, 0))).reshape(NB8, 8 * x1.shape[1])
    BW1 = bigw(W11)
    BW2 = bigw(W12)
    BW3 = bigw(W13)
    Bfc = jnp.einsum('ab,k->akb', eye8, fcW[:, 0]).reshape(256, 8)
    bp11 = bpack(b11)
    bp12 = bpack(b12)
    bp13 = bpack(b13)

    def edge(yp):
        zf = _sc_edge(yp.reshape(NC * N2, H), s_idx, d_idx, N2, H)
        return zf.reshape(NC, NB8, 128)

    degp = _sc_degree(d_idx, N, deg_rows)
    y1, dinv = _tc_stage1(degp, xp, BW1, N, NB8, H)
    z1 = edge(y1)
    y2 = _tc_mid(z1, dinv, bp11, BW2, N, H)
    z2 = edge(y2)
    y3 = _tc_mid(z2, dinv, bp12, BW3, N, H)
    z3 = edge(y3)
    out = _tc_final(z3, dinv, bp13, Bfc, fcb, N, H)
    return out.reshape(N2)[:N].reshape(N, 1)


# speedup vs baseline: 49.2440x; 1.0006x over previous
import functools

import jax
import jax.numpy as jnp
from jax import lax
from jax.experimental import pallas as pl
from jax.experimental.pallas import tpu as pltpu
from jax.experimental.pallas import tpu_sc as plsc

NC = 2
NS = 16
LANES = 16
CHUNK = 128
BN = 4096
PR = BN // 8


def _mesh():
    return plsc.VectorSubcoreMesh(core_axis_name="c", subcore_axis_name="s")


_SC_PARAMS = pltpu.CompilerParams(
    needs_layout_passes=False, use_tc_tiling_on_sc=False)


def _sc_degree(d_idx, N, deg_rows):
    E = d_idx.shape[0]
    NW = NC * NS
    assert E % CHUNK == 0
    T = E // CHUNK
    cpw, extra = divmod(T, NW)
    BATCH = 30
    nbatch, brem = divmod(cpw, BATCH)
    region = deg_rows * 128
    assert N <= region

    @functools.partial(
        pl.kernel,
        out_type=jax.ShapeDtypeStruct((NW, deg_rows, 128), jnp.float32),
        mesh=_mesh(),
        compiler_params=_SC_PARAMS,
        scratch_types=[
            pltpu.VMEM((deg_rows, 128), jnp.float32),
            pltpu.VMEM((BATCH * CHUNK,), jnp.int32),
        ],
    )
    def deg_kernel(d_hbm, out_hbm, deg_v, idx_v):
        c = lax.axis_index("c")
        s = lax.axis_index("s")
        wid = s * NC + c
        zeros16 = jnp.zeros((LANES,), jnp.float32)
        ones16 = jnp.ones((LANES,), jnp.float32)

        def zero_body(i, _):
            deg_v[i // 8, pl.ds((i % 8) * LANES, LANES)] = zeros16
            return 0

        lax.fori_loop(0, region // LANES, zero_body, 0, unroll=8)

        def count(nedge):
            def vec_body(j, _):
                idx = idx_v[pl.ds(j * LANES, LANES)]
                plsc.addupdate_scatter(
                    deg_v,
                    [lax.shift_right_logical(idx, 7),
                     jnp.bitwise_and(idx, 127)],
                    ones16)
                return 0

            lax.fori_loop(0, nedge // LANES, vec_body, 0, unroll=4)

        base = wid * cpw * CHUNK

        def batch_body(i, _):
            pltpu.sync_copy(
                d_hbm.at[pl.ds(base + i * BATCH * CHUNK, BATCH * CHUNK)],
                idx_v)
            count(BATCH * CHUNK)
            return 0

        lax.fori_loop(0, nbatch, batch_body, 0)
        if brem:
            pltpu.sync_copy(
                d_hbm.at[pl.ds(base + nbatch * BATCH * CHUNK, brem * CHUNK)],
                idx_v.at[pl.ds(0, brem * CHUNK)])
            count(brem * CHUNK)
        if extra:
            @pl.when(wid < extra)
            def _():
                pltpu.sync_copy(
                    d_hbm.at[pl.ds((NW * cpw + wid) * CHUNK, CHUNK)],
                    idx_v.at[pl.ds(0, CHUNK)])
                count(CHUNK)
        pltpu.sync_copy(deg_v, out_hbm.at[wid])

    return deg_kernel(d_idx)


def _sc_edge(y_flat, s_idx, d_idx, N2, H):
    E = s_idx.shape[0]
    assert E % CHUNK == 0
    T = E // CHUNK
    cpt, extra = divmod(T, NS)
    rpt = ((N2 // NS) + 7) // 8 * 8
    last_rows = N2 - rpt * (NS - 1)
    assert last_rows > 0 and last_rows % 8 == 0
    D = 6
    pairs = cpt // (2 * D)
    tail = cpt - pairs * 2 * D

    idx_t = pltpu.VMEM((CHUNK,), jnp.int32)
    row_t = pltpu.VMEM((CHUNK, H), jnp.float32)
    scratch = ([pltpu.VMEM_SHARED((N2, H), jnp.float32)]
               + [idx_t] * (4 * D) + [row_t] * (2 * D)
               + [pltpu.SemaphoreType.DMA] * (4 + 2 * D))

    @functools.partial(
        pl.kernel,
        out_type=jax.ShapeDtypeStruct((NC * N2, H), jnp.float32),
        mesh=_mesh(),
        compiler_params=_SC_PARAMS,
        scratch_types=scratch,
    )
    def edge_kernel(y_hbm, s_hbm, d_hbm, z_hbm, z_sh, *sc):
        sA = sc[0:D]
        dA = sc[D:2 * D]
        sB = sc[2 * D:3 * D]
        dB = sc[3 * D:4 * D]
        rA = sc[4 * D:5 * D]
        rB = sc[5 * D:6 * D]
        semiA, semiB, scsA, scsB = sc[6 * D:6 * D + 4]
        gsA = sc[6 * D + 4:7 * D + 4]
        gsB = sc[7 * D + 4:8 * D + 4]

        c = lax.axis_index("c")
        t = lax.axis_index("s")
        cn = c * N2
        yc = y_hbm.at[pl.ds(cn, N2)]
        r0 = t * rpt
        @pl.when(t < NS - 1)
        def _():
            pltpu.sync_copy(y_hbm.at[pl.ds(cn + r0, rpt)],
                            z_sh.at[pl.ds(r0, rpt)])

        @pl.when(t == NS - 1)
        def _():
            pltpu.sync_copy(y_hbm.at[pl.ds(cn + (NS - 1) * rpt, last_rows)],
                            z_sh.at[pl.ds((NS - 1) * rpt, last_rows)])

        plsc.subcore_barrier()

        ebase = t * cpt * CHUNK

        def fire_idx(base, sb, db, sem):
            for k in range(D):
                pltpu.async_copy(
                    s_hbm.at[pl.ds(base + k * CHUNK, CHUNK)], sb[k], sem)
                pltpu.async_copy(
                    d_hbm.at[pl.ds(base + k * CHUNK, CHUNK)], db[k], sem)

        def wait_idx(sb, db, sem):
            for k in range(D):
                pltpu.make_async_copy(
                    s_hbm.at[pl.ds(0, CHUNK)], sb[k], sem).wait()
                pltpu.make_async_copy(
                    s_hbm.at[pl.ds(0, CHUNK)], db[k], sem).wait()

        def do_group(sb, db, rr, gs, scs):
            for k in range(D):
                pltpu.async_copy(yc.at[sb[k]], rr[k], gs[k])
            for k in range(D):
                pltpu.make_async_copy(
                    yc.at[pl.ds(0, CHUNK)], rr[k], gs[k]).wait()
                pltpu.async_copy(rr[k], z_sh.at[db[k]], scs, add=True)
            for k in range(D):
                pltpu.make_async_copy(rr[k], z_sh.at[db[k]], scs).wait()

        fire_idx(ebase, sA, dA, semiA)

        def pair_body(p, _):
            gA = 2 * p
            wait_idx(sA, dA, semiA)
            fire_idx(ebase + (gA + 1) * D * CHUNK, sB, dB, semiB)
            do_group(sA, dA, rA, gsA, scsA)
            wait_idx(sB, dB, semiB)

            @pl.when(p + 1 < pairs)
            def _():
                fire_idx(ebase + (gA + 2) * D * CHUNK, sA, dA, semiA)

            do_group(sB, dB, rB, gsB, scsB)
            return 0

        lax.fori_loop(0, pairs, pair_body, 0)

        def step(b):
            pltpu.sync_copy(s_hbm.at[b], sA[0])
            pltpu.sync_copy(d_hbm.at[b], dA[0])
            pltpu.async_copy(yc.at[sA[0]], rA[0], gsA[0]).wait()
            pltpu.sync_copy(rA[0], z_sh.at[dA[0]], add=True)

        for i in range(tail):
            step(pl.ds(ebase + (pairs * 2 * D + i) * CHUNK, CHUNK))
        if extra:
            @pl.when(t < extra)
            def _():
                step(pl.ds((NS * cpt + t) * CHUNK, CHUNK))

        plsc.subcore_barrier()

        @pl.when(t < NS - 1)
        def _():
            pltpu.sync_copy(z_sh.at[pl.ds(r0, rpt)],
                            z_hbm.at[pl.ds(cn + r0, rpt)])

        @pl.when(t == NS - 1)
        def _():
            pltpu.sync_copy(z_sh.at[pl.ds((NS - 1) * rpt, last_rows)],
                            z_hbm.at[pl.ds(cn + (NS - 1) * rpt, last_rows)])

    return edge_kernel(y_flat, s_idx, d_idx)


def _dvp(dinv):
    rep = jnp.broadcast_to(dinv[:, None], (BN, 32))
    return rep.reshape(PR, 8, 32)[:, :, :16].reshape(PR, 128)


def _tc_stage1(degp, xp, BW1, N, NB8, H):
    NW = degp.shape[0]
    degp = degp.reshape(NW, degp.shape[1] * 128)
    K = xp.shape[1]
    grid = pl.cdiv(N, BN)
    assert degp.shape[1] == grid * BN

    def body(degp_ref, xp_ref, w_ref, y_ref, dinv_ref):
        dinv = lax.rsqrt(1.0 + jnp.sum(degp_ref[...], axis=0))
        dvp = _dvp(dinv)
        Y = jnp.dot(xp_ref[...], w_ref[...],
                    preferred_element_type=jnp.float32)
        y_ref[0] = Y[:, :128] * dvp
        y_ref[1] = Y[:, 128:] * dvp
        dinv_ref[...] = dinv

    return pl.pallas_call(
        body,
        grid=(grid,),
        in_specs=[
            pl.BlockSpec((NW, BN), lambda i: (0, i)),
            pl.BlockSpec((PR, K), lambda i: (i, 0)),
            pl.BlockSpec((K, 256), lambda i: (0, 0)),
        ],
        out_specs=[
            pl.BlockSpec((NC, PR, 128), lambda i: (0, i, 0)),
            pl.BlockSpec((BN,), lambda i: (i,)),
        ],
        out_shape=[
            jax.ShapeDtypeStruct((NC, NB8, 128), jnp.float32),
            jax.ShapeDtypeStruct((N,), jnp.float32),
        ],
    )(degp, xp, BW1)


def _tc_mid(z, dinv, bp, BW, N, H):
    grid = pl.cdiv(N, BN)
    NB8 = z.shape[1]

    def body(z_ref, dinv_ref, bp_ref, w_ref, y_ref):
        dvp = _dvp(dinv_ref[...])
        h0 = jnp.maximum(z_ref[0] * dvp + bp_ref[0][None, :], 0.0)
        h1 = jnp.maximum(z_ref[1] * dvp + bp_ref[1][None, :], 0.0)
        Hp = jnp.concatenate([h0, h1], axis=1)
        Y = jnp.dot(Hp, w_ref[...], preferred_element_type=jnp.float32)
        y_ref[0] = Y[:, :128] * dvp
        y_ref[1] = Y[:, 128:] * dvp

    return pl.pallas_call(
        body,
        grid=(grid,),
        in_specs=[
            pl.BlockSpec((NC, PR, 128), lambda i: (0, i, 0)),
            pl.BlockSpec((BN,), lambda i: (i,)),
            pl.BlockSpec((2, 128), lambda i: (0, 0)),
            pl.BlockSpec((256, 256), lambda i: (0, 0)),
        ],
        out_specs=pl.BlockSpec((NC, PR, 128), lambda i: (0, i, 0)),
        out_shape=jax.ShapeDtypeStruct((NC, NB8, 128), jnp.float32),
    )(z, dinv, bp, BW)


def _tc_final(z, dinv, bp, Bfc, fcb, N, H):
    grid = pl.cdiv(N, BN)

    def body(z_ref, dinv_ref, bp_ref, w_ref, fcb_ref, o_ref):
        dvp = _dvp(dinv_ref[...])
        h0 = jnp.maximum(z_ref[0] * dvp + bp_ref[0][None, :], 0.0)
        h1 = jnp.maximum(z_ref[1] * dvp + bp_ref[1][None, :], 0.0)
        Hp = jnp.concatenate([h0, h1], axis=1)
        o8 = jnp.dot(Hp, w_ref[...], preferred_element_type=jnp.float32)
        o_ref[...] = o8 + fcb_ref[0]

    NB8 = z.shape[1]
    return pl.pallas_call(
        body,
        grid=(grid,),
        in_specs=[
            pl.BlockSpec((NC, PR, 128), lambda i: (0, i, 0)),
            pl.BlockSpec((BN,), lambda i: (i,)),
            pl.BlockSpec((2, 128), lambda i: (0, 0)),
            pl.BlockSpec((256, 8), lambda i: (0, 0)),
            pl.BlockSpec((1,), lambda i: (0,)),
        ],
        out_specs=pl.BlockSpec((PR, 8), lambda i: (i, 0)),
        out_shape=jax.ShapeDtypeStruct((NB8, 8), jnp.float32),
    )(z, dinv, bp, Bfc, fcb)


def kernel(x1, edge_index1, x2, edge_index2, x3, edge_index3,
           W11, b11, W12, b12, W13, b13,
           W21, b21, W22, b22, W23, b23,
           W31, b31, W32, b32, W33, b33,
           fcW, fcb):
    N = x1.shape[0]
    H = W11.shape[1] // NC
    grid = pl.cdiv(N, BN)
    deg_rows = grid * (BN // 128)
    NB8 = ((N + 7) // 8 + 7) // 8 * 8
    N2 = NB8 * (128 // H)

    s_idx = edge_index1[0]
    d_idx = edge_index1[1]

    eye8 = jnp.eye(8, dtype=jnp.float32)

    def bigw(W):
        Wr = W.reshape(2, H, 2, H)
        return jnp.einsum('ab,xicf->xaicbf', eye8, Wr).reshape(256, 256)

    def bpack(b):
        return jnp.tile(b.reshape(2, 1, H), (1, 8, 1)).reshape(2, 128)

    W11r = W11.reshape(x1.shape[1], 2, H)
    BW1 = jnp.einsum('ab,jcf->ajcbf', eye8, W11r).reshape(
        8 * x1.shape[1], 256)
    xp = jnp.pad(x1, ((0, N2 - N), (0, 0))).reshape(NB8, 8 * x1.shape[1])
    BW2 = bigw(W12)
    BW3 = bigw(W13)
    Bfc = jnp.einsum('ab,xi->xaib', eye8,
                     fcW[:, 0].reshape(2, H)).reshape(256, 8)
    bp11 = bpack(b11)
    bp12 = bpack(b12)
    bp13 = bpack(b13)

    def edge(yp):
        zf = _sc_edge(yp.reshape(NC * N2, H), s_idx, d_idx, N2, H)
        return zf.reshape(NC, NB8, 128)

    degp = _sc_degree(d_idx, N, deg_rows)
    y1, dinv = _tc_stage1(degp, xp, BW1, N, NB8, H)
    z1 = edge(y1)
    y2 = _tc_mid(z1, dinv, bp11, BW2, N, H)
    z2 = edge(y2)
    y3 = _tc_mid(z2, dinv, bp12, BW3, N, H)
    z3 = edge(y3)
    out = _tc_final(z3, dinv, bp13, Bfc, fcb, N, H)
    return out.reshape(N2)[:N].reshape(N, 1)


# speedup vs baseline: 49.5949x; 1.0071x over previous
import functools

import jax
import jax.numpy as jnp
from jax import lax
from jax.experimental import pallas as pl
from jax.experimental.pallas import tpu as pltpu
from jax.experimental.pallas import tpu_sc as plsc

NC = 2
NS = 16
LANES = 16
CHUNK = 128
BN = 8192
PR = BN // 8


def _mesh():
    return plsc.VectorSubcoreMesh(core_axis_name="c", subcore_axis_name="s")


_SC_PARAMS = pltpu.CompilerParams(
    needs_layout_passes=False, use_tc_tiling_on_sc=False)


def _sc_degree(d_idx, N, deg_rows):
    E = d_idx.shape[0]
    NW = NC * NS
    assert E % CHUNK == 0
    T = E // CHUNK
    cpw, extra = divmod(T, NW)
    BATCH = 30
    nbatch, brem = divmod(cpw, BATCH)
    region = deg_rows * 128
    assert N <= region

    @functools.partial(
        pl.kernel,
        out_type=jax.ShapeDtypeStruct((NW, deg_rows, 128), jnp.float32),
        mesh=_mesh(),
        compiler_params=_SC_PARAMS,
        scratch_types=[
            pltpu.VMEM((deg_rows, 128), jnp.float32),
            pltpu.VMEM((BATCH * CHUNK,), jnp.int32),
        ],
    )
    def deg_kernel(d_hbm, out_hbm, deg_v, idx_v):
        c = lax.axis_index("c")
        s = lax.axis_index("s")
        wid = s * NC + c
        zeros16 = jnp.zeros((LANES,), jnp.float32)
        ones16 = jnp.ones((LANES,), jnp.float32)

        def zero_body(i, _):
            deg_v[i // 8, pl.ds((i % 8) * LANES, LANES)] = zeros16
            return 0

        lax.fori_loop(0, region // LANES, zero_body, 0, unroll=8)

        def count(nedge):
            def vec_body(j, _):
                idx = idx_v[pl.ds(j * LANES, LANES)]
                plsc.addupdate_scatter(
                    deg_v,
                    [lax.shift_right_logical(idx, 7),
                     jnp.bitwise_and(idx, 127)],
                    ones16)
                return 0

            lax.fori_loop(0, nedge // LANES, vec_body, 0, unroll=4)

        base = wid * cpw * CHUNK

        def batch_body(i, _):
            pltpu.sync_copy(
                d_hbm.at[pl.ds(base + i * BATCH * CHUNK, BATCH * CHUNK)],
                idx_v)
            count(BATCH * CHUNK)
            return 0

        lax.fori_loop(0, nbatch, batch_body, 0)
        if brem:
            pltpu.sync_copy(
                d_hbm.at[pl.ds(base + nbatch * BATCH * CHUNK, brem * CHUNK)],
                idx_v.at[pl.ds(0, brem * CHUNK)])
            count(brem * CHUNK)
        if extra:
            @pl.when(wid < extra)
            def _():
                pltpu.sync_copy(
                    d_hbm.at[pl.ds((NW * cpw + wid) * CHUNK, CHUNK)],
                    idx_v.at[pl.ds(0, CHUNK)])
                count(CHUNK)
        pltpu.sync_copy(deg_v, out_hbm.at[wid])

    return deg_kernel(d_idx)


def _sc_edge(y_flat, s_idx, d_idx, N2, H):
    E = s_idx.shape[0]
    assert E % CHUNK == 0
    T = E // CHUNK
    cpt, extra = divmod(T, NS)
    rpt = ((N2 // NS) + 7) // 8 * 8
    last_rows = N2 - rpt * (NS - 1)
    assert last_rows > 0 and last_rows % 8 == 0
    D = 6
    pairs = cpt // (2 * D)
    tail = cpt - pairs * 2 * D

    idx_t = pltpu.VMEM((CHUNK,), jnp.int32)
    row_t = pltpu.VMEM((CHUNK, H), jnp.float32)
    scratch = ([pltpu.VMEM_SHARED((N2, H), jnp.float32)]
               + [idx_t] * (4 * D) + [row_t] * (2 * D)
               + [pltpu.SemaphoreType.DMA] * (4 + 2 * D))

    @functools.partial(
        pl.kernel,
        out_type=jax.ShapeDtypeStruct((NC * N2, H), jnp.float32),
        mesh=_mesh(),
        compiler_params=_SC_PARAMS,
        scratch_types=scratch,
    )
    def edge_kernel(y_hbm, s_hbm, d_hbm, z_hbm, z_sh, *sc):
        sA = sc[0:D]
        dA = sc[D:2 * D]
        sB = sc[2 * D:3 * D]
        dB = sc[3 * D:4 * D]
        rA = sc[4 * D:5 * D]
        rB = sc[5 * D:6 * D]
        semiA, semiB, scsA, scsB = sc[6 * D:6 * D + 4]
        gsA = sc[6 * D + 4:7 * D + 4]
        gsB = sc[7 * D + 4:8 * D + 4]

        c = lax.axis_index("c")
        t = lax.axis_index("s")
        cn = c * N2
        yc = y_hbm.at[pl.ds(cn, N2)]
        r0 = t * rpt
        @pl.when(t < NS - 1)
        def _():
            pltpu.sync_copy(y_hbm.at[pl.ds(cn + r0, rpt)],
                            z_sh.at[pl.ds(r0, rpt)])

        @pl.when(t == NS - 1)
        def _():
            pltpu.sync_copy(y_hbm.at[pl.ds(cn + (NS - 1) * rpt, last_rows)],
                            z_sh.at[pl.ds((NS - 1) * rpt, last_rows)])

        plsc.subcore_barrier()

        ebase = t * cpt * CHUNK

        def fire_idx(base, sb, db, sem):
            for k in range(D):
                pltpu.async_copy(
                    s_hbm.at[pl.ds(base + k * CHUNK, CHUNK)], sb[k], sem)
                pltpu.async_copy(
                    d_hbm.at[pl.ds(base + k * CHUNK, CHUNK)], db[k], sem)

        def wait_idx(sb, db, sem):
            for k in range(D):
                pltpu.make_async_copy(
                    s_hbm.at[pl.ds(0, CHUNK)], sb[k], sem).wait()
                pltpu.make_async_copy(
                    s_hbm.at[pl.ds(0, CHUNK)], db[k], sem).wait()

        def do_group(sb, db, rr, gs, scs):
            for k in range(D):
                pltpu.async_copy(yc.at[sb[k]], rr[k], gs[k])
            for k in range(D):
                pltpu.make_async_copy(
                    yc.at[pl.ds(0, CHUNK)], rr[k], gs[k]).wait()
                pltpu.async_copy(rr[k], z_sh.at[db[k]], scs, add=True)
            for k in range(D):
                pltpu.make_async_copy(rr[k], z_sh.at[db[k]], scs).wait()

        fire_idx(ebase, sA, dA, semiA)

        def pair_body(p, _):
            gA = 2 * p
            wait_idx(sA, dA, semiA)
            fire_idx(ebase + (gA + 1) * D * CHUNK, sB, dB, semiB)
            do_group(sA, dA, rA, gsA, scsA)
            wait_idx(sB, dB, semiB)

            @pl.when(p + 1 < pairs)
            def _():
                fire_idx(ebase + (gA + 2) * D * CHUNK, sA, dA, semiA)

            do_group(sB, dB, rB, gsB, scsB)
            return 0

        lax.fori_loop(0, pairs, pair_body, 0)

        def step(b):
            pltpu.sync_copy(s_hbm.at[b], sA[0])
            pltpu.sync_copy(d_hbm.at[b], dA[0])
            pltpu.async_copy(yc.at[sA[0]], rA[0], gsA[0]).wait()
            pltpu.sync_copy(rA[0], z_sh.at[dA[0]], add=True)

        for i in range(tail):
            step(pl.ds(ebase + (pairs * 2 * D + i) * CHUNK, CHUNK))
        if extra:
            @pl.when(t < extra)
            def _():
                step(pl.ds((NS * cpt + t) * CHUNK, CHUNK))

        plsc.subcore_barrier()

        @pl.when(t < NS - 1)
        def _():
            pltpu.sync_copy(z_sh.at[pl.ds(r0, rpt)],
                            z_hbm.at[pl.ds(cn + r0, rpt)])

        @pl.when(t == NS - 1)
        def _():
            pltpu.sync_copy(z_sh.at[pl.ds((NS - 1) * rpt, last_rows)],
                            z_hbm.at[pl.ds(cn + (NS - 1) * rpt, last_rows)])

    return edge_kernel(y_flat, s_idx, d_idx)


def _dvp(dinv):
    rep = jnp.broadcast_to(dinv[:, None], (BN, 32))
    return rep.reshape(PR, 8, 32)[:, :, :16].reshape(PR, 128)


def _tc_stage1(degp, xp, BW1, N, NB8, H):
    NW = degp.shape[0]
    degp = degp.reshape(NW, degp.shape[1] * 128)
    K = xp.shape[1]
    grid = pl.cdiv(N, BN)
    assert degp.shape[1] == grid * BN

    def body(degp_ref, xp_ref, w_ref, y_ref, dinv_ref):
        dinv = lax.rsqrt(1.0 + jnp.sum(degp_ref[...], axis=0))
        dvp = _dvp(dinv)
        Y = jnp.dot(xp_ref[...], w_ref[...],
                    preferred_element_type=jnp.float32)
        y_ref[0] = Y[:, :128] * dvp
        y_ref[1] = Y[:, 128:] * dvp
        dinv_ref[...] = dinv

    return pl.pallas_call(
        body,
        grid=(grid,),
        in_specs=[
            pl.BlockSpec((NW, BN), lambda i: (0, i)),
            pl.BlockSpec((PR, K), lambda i: (i, 0)),
            pl.BlockSpec((K, 256), lambda i: (0, 0)),
        ],
        out_specs=[
            pl.BlockSpec((NC, PR, 128), lambda i: (0, i, 0)),
            pl.BlockSpec((BN,), lambda i: (i,)),
        ],
        out_shape=[
            jax.ShapeDtypeStruct((NC, NB8, 128), jnp.float32),
            jax.ShapeDtypeStruct((N,), jnp.float32),
        ],
    )(degp, xp, BW1)


def _tc_mid(z, dinv, bp, BW, N, H):
    grid = pl.cdiv(N, BN)
    NB8 = z.shape[1]

    def body(z_ref, dinv_ref, bp_ref, w_ref, y_ref):
        dvp = _dvp(dinv_ref[...])
        h0 = jnp.maximum(z_ref[0] * dvp + bp_ref[0][None, :], 0.0)
        h1 = jnp.maximum(z_ref[1] * dvp + bp_ref[1][None, :], 0.0)
        Hp = jnp.concatenate([h0, h1], axis=1)
        Y = jnp.dot(Hp, w_ref[...], preferred_element_type=jnp.float32)
        y_ref[0] = Y[:, :128] * dvp
        y_ref[1] = Y[:, 128:] * dvp

    return pl.pallas_call(
        body,
        grid=(grid,),
        in_specs=[
            pl.BlockSpec((NC, PR, 128), lambda i: (0, i, 0)),
            pl.BlockSpec((BN,), lambda i: (i,)),
            pl.BlockSpec((2, 128), lambda i: (0, 0)),
            pl.BlockSpec((256, 256), lambda i: (0, 0)),
        ],
        out_specs=pl.BlockSpec((NC, PR, 128), lambda i: (0, i, 0)),
        out_shape=jax.ShapeDtypeStruct((NC, NB8, 128), jnp.float32),
    )(z, dinv, bp, BW)


def _tc_final(z, dinv, bp, Bfc, fcb, N, H):
    grid = pl.cdiv(N, BN)

    def body(z_ref, dinv_ref, bp_ref, w_ref, fcb_ref, o_ref):
        dvp = _dvp(dinv_ref[...])
        h0 = jnp.maximum(z_ref[0] * dvp + bp_ref[0][None, :], 0.0)
        h1 = jnp.maximum(z_ref[1] * dvp + bp_ref[1][None, :], 0.0)
        Hp = jnp.concatenate([h0, h1], axis=1)
        o8 = jnp.dot(Hp, w_ref[...], preferred_element_type=jnp.float32)
        o_ref[...] = o8 + fcb_ref[0]

    NB8 = z.shape[1]
    return pl.pallas_call(
        body,
        grid=(grid,),
        in_specs=[
            pl.BlockSpec((NC, PR, 128), lambda i: (0, i, 0)),
            pl.BlockSpec((BN,), lambda i: (i,)),
            pl.BlockSpec((2, 128), lambda i: (0, 0)),
            pl.BlockSpec((256, 8), lambda i: (0, 0)),
            pl.BlockSpec((1,), lambda i: (0,)),
        ],
        out_specs=pl.BlockSpec((PR, 8), lambda i: (i, 0)),
        out_shape=jax.ShapeDtypeStruct((NB8, 8), jnp.float32),
    )(z, dinv, bp, Bfc, fcb)


def kernel(x1, edge_index1, x2, edge_index2, x3, edge_index3,
           W11, b11, W12, b12, W13, b13,
           W21, b21, W22, b22, W23, b23,
           W31, b31, W32, b32, W33, b33,
           fcW, fcb):
    N = x1.shape[0]
    H = W11.shape[1] // NC
    grid = pl.cdiv(N, BN)
    deg_rows = grid * (BN // 128)
    NB8 = ((N + 7) // 8 + 7) // 8 * 8
    N2 = NB8 * (128 // H)

    s_idx = edge_index1[0]
    d_idx = edge_index1[1]

    eye8 = jnp.eye(8, dtype=jnp.float32)

    def bigw(W):
        Wr = W.reshape(2, H, 2, H)
        return jnp.einsum('ab,xicf->xaicbf', eye8, Wr).reshape(256, 256)

    def bpack(b):
        return jnp.tile(b.reshape(2, 1, H), (1, 8, 1)).reshape(2, 128)

    W11r = W11.reshape(x1.shape[1], 2, H)
    BW1 = jnp.einsum('ab,jcf->ajcbf', eye8, W11r).reshape(
        8 * x1.shape[1], 256)
    xp = jnp.pad(x1, ((0, N2 - N), (0, 0))).reshape(NB8, 8 * x1.shape[1])
    BW2 = bigw(W12)
    BW3 = bigw(W13)
    Bfc = jnp.einsum('ab,xi->xaib', eye8,
                     fcW[:, 0].reshape(2, H)).reshape(256, 8)
    bp11 = bpack(b11)
    bp12 = bpack(b12)
    bp13 = bpack(b13)

    def edge(yp):
        zf = _sc_edge(yp.reshape(NC * N2, H), s_idx, d_idx, N2, H)
        return zf.reshape(NC, NB8, 128)

    degp = _sc_degree(d_idx, N, deg_rows)
    y1, dinv = _tc_stage1(degp, xp, BW1, N, NB8, H)
    z1 = edge(y1)
    y2 = _tc_mid(z1, dinv, bp11, BW2, N, H)
    z2 = edge(y2)
    y3 = _tc_mid(z2, dinv, bp12, BW3, N, H)
    z3 = edge(y3)
    out = _tc_final(z3, dinv, bp13, Bfc, fcb, N, H)
    return out.reshape(N2)[:N].reshape(N, 1)
